# Initial kernel scaffold; baseline (speedup 1.0000x reference)
import functools

import jax
import jax.numpy as jnp
from jax import lax
from jax.experimental import pallas as pl
from jax.experimental.pallas import tpu as pltpu
from jax.experimental.pallas import tpu_sc as plsc

N = 50000
E = 800000
H = 4
NPAD = 50016
EPAD = 819200
NW = 32
CH = 1024
CHG = 512
BN = 2000
BE = 2000
ROWS_SUB = NPAD // 16


def _mmatt_body(h_ref, w_ref, as_ref, ad_ref, xw_ref, als_ref, ald_ref,
                maxs_ref):
    i = pl.program_id(0)
    xw = jnp.dot(h_ref[...], w_ref[...], preferred_element_type=jnp.float32)
    xw_ref[...] = xw
    als = jnp.dot(xw, as_ref[...], preferred_element_type=jnp.float32)
    ald = jnp.dot(xw, ad_ref[...], preferred_element_type=jnp.float32)
    als_ref[...] = als
    ald_ref[...] = ald

    @pl.when(i == 0)
    def _():
        maxs_ref[...] = jnp.full_like(maxs_ref[...], -jnp.inf)

    maxs_ref[...] = jnp.maximum(maxs_ref[...],
                                jnp.max(als, axis=0, keepdims=True))


def _mmatt(h, W, as_mat, ad_mat, C):
    n_in = h.shape[1]
    return pl.pallas_call(
        _mmatt_body,
        grid=(N // BN,),
        in_specs=[
            pl.BlockSpec((BN, n_in), lambda i: (i, 0)),
            pl.BlockSpec((n_in, H * C), lambda i: (0, 0)),
            pl.BlockSpec((H * C, H), lambda i: (0, 0)),
            pl.BlockSpec((H * C, H), lambda i: (0, 0)),
        ],
        out_specs=[
            pl.BlockSpec((BN, H * C), lambda i: (i, 0)),
            pl.BlockSpec((BN, H), lambda i: (i, 0)),
            pl.BlockSpec((BN, H), lambda i: (i, 0)),
            pl.BlockSpec((1, H), lambda i: (0, 0)),
        ],
        out_shape=[
            jax.ShapeDtypeStruct((N, H * C), jnp.float32),
            jax.ShapeDtypeStruct((N, H), jnp.float32),
            jax.ShapeDtypeStruct((N, H), jnp.float32),
            jax.ShapeDtypeStruct((1, H), jnp.float32),
        ],
    )(h, W, as_mat, ad_mat)


def _assemble_body(als_ref, ald_ref, maxs_ref, at_ref, slw_ref):
    als = als_ref[...]
    ald = ald_ref[...]
    t = maxs_ref[...] + ald
    bnd = jnp.maximum(t, 0.2 * t)
    a = als + ald
    a = jnp.maximum(a, 0.2 * a)
    slw_ref[...] = jnp.exp(a - bnd)
    at_ref[...] = jnp.concatenate(
        [als, ald, bnd, jnp.zeros_like(als)], axis=1)


def _assemble(als, ald, maxs):
    return pl.pallas_call(
        _assemble_body,
        grid=(N // BN,),
        in_specs=[
            pl.BlockSpec((BN, H), lambda i: (i, 0)),
            pl.BlockSpec((BN, H), lambda i: (i, 0)),
            pl.BlockSpec((1, H), lambda i: (0, 0)),
        ],
        out_specs=[
            pl.BlockSpec((BN, 16), lambda i: (i, 0)),
            pl.BlockSpec((BN, H), lambda i: (i, 0)),
        ],
        out_shape=[
            jax.ShapeDtypeStruct((N, 16), jnp.float32),
            jax.ShapeDtypeStruct((N, H), jnp.float32),
        ],
    )(als, ald, maxs)


def _combine_body(acc0_ref, acc1_ref, asum0_ref, asum1_ref, slw_ref, xw_ref,
                  b_ref, out_ref, *, C, concat_halves):
    slw = slw_ref[...]
    out = jnp.zeros(out_ref.shape, jnp.float32)
    for h in range(H):
        if concat_halves:
            num = jnp.concatenate([acc0_ref[h], acc1_ref[h]], axis=1)
        else:
            num = acc0_ref[h] + acc1_ref[h]
        num = num + slw[:, h:h + 1] * xw_ref[:, h * C:(h + 1) * C]
        den = (asum0_ref[:, h:h + 1] + asum1_ref[:, h:h + 1]
               + slw[:, h:h + 1] + 1e-16)
        out = out + num / den
    out_ref[...] = out * (1.0 / H) + b_ref[...]


def _combine(acc0, acc1, asum0, asum1, slw, xw, bias, C, concat_halves):
    cp = acc0.shape[-1]
    body = functools.partial(_combine_body, C=C, concat_halves=concat_halves)
    return pl.pallas_call(
        body,
        grid=(N // BN,),
        in_specs=[
            pl.BlockSpec((H, BN, cp), lambda i: (0, i, 0)),
            pl.BlockSpec((H, BN, cp), lambda i: (0, i, 0)),
            pl.BlockSpec((BN, H), lambda i: (i, 0)),
            pl.BlockSpec((BN, H), lambda i: (i, 0)),
            pl.BlockSpec((BN, H), lambda i: (i, 0)),
            pl.BlockSpec((BN, H * C), lambda i: (i, 0)),
            pl.BlockSpec((1, C), lambda i: (0, 0)),
        ],
        out_specs=pl.BlockSpec((BN, C), lambda i: (i, 0)),
        out_shape=jax.ShapeDtypeStruct((N, C), jnp.float32),
    )(acc0, acc1, asum0, asum1, slw, xw, bias)


def _pq_body(h_ref, m_ref, p_ref, q_ref):
    pq = jnp.dot(h_ref[...], m_ref[...], preferred_element_type=jnp.float32)
    p_ref[...] = pq[:, :64]
    q_ref[...] = pq[:, 64:]


def _pq(h3, m1pq):
    return pl.pallas_call(
        _pq_body,
        grid=(N // BN,),
        in_specs=[
            pl.BlockSpec((BN, 64), lambda i: (i, 0)),
            pl.BlockSpec((64, 128), lambda i: (0, 0)),
        ],
        out_specs=[
            pl.BlockSpec((BN, 64), lambda i: (i, 0)),
            pl.BlockSpec((BN, 64), lambda i: (i, 0)),
        ],
        out_shape=[
            jax.ShapeDtypeStruct((N, 64), jnp.float32),
            jax.ShapeDtypeStruct((N, 64), jnp.float32),
        ],
    )(h3, m1pq)


def _mlp_body(s_ref, ea_ref, m1c_ref, mb1_ref, m2_ref, mb2_ref, m3_ref,
              mb3_ref, out_ref):
    z = (s_ref[...]
         + jnp.dot(ea_ref[...], m1c_ref[...],
                   preferred_element_type=jnp.float32)
         + mb1_ref[...])
    z = jnp.maximum(z, 0.12 * z)
    z = jnp.dot(z, m2_ref[...], preferred_element_type=jnp.float32) + mb2_ref[...]
    z = jnp.maximum(z, 0.12 * z)
    t = jnp.dot(z, m3_ref[...], preferred_element_type=jnp.float32) + mb3_ref[...]
    out_ref[...] = jax.nn.sigmoid(t)


def _mlp(s, ea, m1c, mb1, m2, mb2, m3, mb3):
    return pl.pallas_call(
        _mlp_body,
        grid=(E // BE,),
        in_specs=[
            pl.BlockSpec((BE, 64), lambda i: (i, 0)),
            pl.BlockSpec((BE, 10), lambda i: (i, 0)),
            pl.BlockSpec((10, 64), lambda i: (0, 0)),
            pl.BlockSpec((1, 64), lambda i: (0, 0)),
            pl.BlockSpec((64, 16), lambda i: (0, 0)),
            pl.BlockSpec((1, 16), lambda i: (0, 0)),
            pl.BlockSpec((16, 1), lambda i: (0, 0)),
            pl.BlockSpec((1, 1), lambda i: (0, 0)),
        ],
        out_specs=pl.BlockSpec((BE, 1), lambda i: (i, 0)),
        out_shape=jax.ShapeDtypeStruct((E, 1), jnp.float32),
    )(s, ea, m1c, mb1, m2, mb2, m3, mb3)


_MESH = plsc.VectorSubcoreMesh(core_axis_name="c", subcore_axis_name="s")


def _full16(v):
    return jnp.full((16,), v, jnp.int32)


@functools.lru_cache(maxsize=None)
def _make_sc_w():
    ew = EPAD // NW
    nchunks = ew // CH

    @functools.partial(
        pl.kernel,
        mesh=_MESH,
        out_type=[
            jax.ShapeDtypeStruct((H, EPAD), jnp.float32),
            jax.ShapeDtypeStruct((2, NPAD, 16), jnp.float32),
        ],
        scratch_types=[
            pltpu.VMEM((8, 128), jnp.int32),
            pltpu.VMEM((8, 128), jnp.int32),
            pltpu.VMEM((CH, 16), jnp.float32),
            pltpu.VMEM((CH, 16), jnp.float32),
            pltpu.VMEM((CH, 16), jnp.float32),
            pltpu.VMEM((H, CH), jnp.float32),
            pltpu.VMEM_SHARED((NPAD, 16), jnp.float32),
            pltpu.SemaphoreType.DMA,
        ],
    )
    def sc_w(at_hbm, src_hbm, dst_hbm, zeros_hbm, wt_hbm, asum_hbm,
             srcv, dstv, at_src, at_dst, wpad, wlin, asum_sh, sem):
        cid = lax.axis_index("c")
        sid = lax.axis_index("s")
        pltpu.sync_copy(zeros_hbm.at[pl.ds(sid * ROWS_SUB, ROWS_SUB)],
                        asum_sh.at[pl.ds(sid * ROWS_SUB, ROWS_SUB)])
        zv = jnp.zeros((16,), jnp.float32)

        def _zero(r, carry):
            wpad[r, :] = zv
            return carry

        lax.fori_loop(0, CH, _zero, 0)
        plsc.subcore_barrier()

        base0 = (cid * 16 + sid) * ew
        iota = lax.iota(jnp.int32, 16)

        def _chunk(ci, carry):
            ebase = base0 + ci * CH
            eb128 = ebase // 128
            pltpu.sync_copy(src_hbm.at[pl.ds(eb128, 8)], srcv)
            pltpu.sync_copy(dst_hbm.at[pl.ds(eb128, 8)], dstv)
            cps = [pltpu.async_copy(at_hbm.at[srcv.at[j]],
                                    at_src.at[pl.ds(j * 128, 128)], sem)
                   for j in range(8)]
            cps += [pltpu.async_copy(at_hbm.at[dstv.at[j]],
                                     at_dst.at[pl.ds(j * 128, 128)], sem)
                    for j in range(8)]
            for cp in cps:
                cp.wait()

            def _grp(j, carry2):
                rows = iota + j * 16
                for h in range(H):
                    als = plsc.load_gather(at_src, [rows, _full16(h)])
                    ald = plsc.load_gather(at_dst, [rows, _full16(4 + h)])
                    bnd = plsc.load_gather(at_dst, [rows, _full16(8 + h)])
                    a = als + ald
                    a = jnp.maximum(a, 0.2 * a)
                    w = jnp.exp(a - bnd)
                    wlin[h, pl.ds(j * 16, 16)] = w
                    plsc.store_scatter(wpad, [rows, _full16(h)], w)
                return carry2

            lax.fori_loop(0, CH // 16, _grp, 0)
            for h in range(H):
                pltpu.sync_copy(wlin.at[h], wt_hbm.at[h, pl.ds(ebase, CH)])
            for j in range(8):
                pltpu.sync_copy(wpad.at[pl.ds(j * 128, 128)],
                                asum_sh.at[dstv.at[j]], add=True)
            return carry

        lax.fori_loop(0, nchunks, _chunk, 0)
        plsc.subcore_barrier()
        pltpu.sync_copy(asum_sh.at[pl.ds(sid * ROWS_SUB, ROWS_SUB)],
                        asum_hbm.at[cid, pl.ds(sid * ROWS_SUB, ROWS_SUB)])

    return sc_w


@functools.lru_cache(maxsize=None)
def _make_sc_m(cp, split_channels, kmul):
    ew = EPAD // 16 if split_channels else EPAD // NW
    nchunks = ew // CH

    @functools.partial(
        pl.kernel,
        mesh=_MESH,
        out_type=jax.ShapeDtypeStruct((H, 2, NPAD, cp), jnp.float32),
        scratch_types=[
            pltpu.VMEM((8, 128), jnp.int32),
            pltpu.VMEM((8, 128), jnp.int32),
            pltpu.VMEM((8, 128), jnp.int32),
            pltpu.VMEM((CH, cp), jnp.float32),
            pltpu.VMEM((CH,), jnp.float32),
            pltpu.VMEM_SHARED((NPAD, cp), jnp.float32),
            pltpu.SemaphoreType.DMA,
        ],
    )
    def sc_m(xwt_hbm, src_hbm, dst_hbm, wt_hbm, zeros_hbm, out_hbm,
             srcv, gidx, dstv, rows, wv, acc_sh, sem):
        cid = lax.axis_index("c")
        sid = lax.axis_index("s")
        if split_channels:
            base0 = sid * ew
            goff = (kmul // H) * cid
        else:
            base0 = (cid * 16 + sid) * ew
            goff = cid * 0
        hstep = kmul // H

        for h in range(H):
            pltpu.sync_copy(zeros_hbm.at[pl.ds(sid * ROWS_SUB, ROWS_SUB)],
                            acc_sh.at[pl.ds(sid * ROWS_SUB, ROWS_SUB)])
            plsc.subcore_barrier()

            def _chunk(ci, carry, h=h):
                ebase = base0 + ci * CH
                eb128 = ebase // 128
                pltpu.sync_copy(src_hbm.at[pl.ds(eb128, 8)], srcv)
                pltpu.sync_copy(dst_hbm.at[pl.ds(eb128, 8)], dstv)

                def _gi(t, carry2):
                    r = t >> 3
                    c = (t & 7) * 16
                    v = srcv[r, pl.ds(c, 16)]
                    gidx[r, pl.ds(c, 16)] = v * kmul + (h * hstep + goff)
                    return carry2

                lax.fori_loop(0, CH // 16, _gi, 0)
                cps = [pltpu.async_copy(xwt_hbm.at[gidx.at[j]],
                                        rows.at[pl.ds(j * 128, 128)], sem)
                       for j in range(8)]
                pltpu.sync_copy(wt_hbm.at[h, pl.ds(ebase, CH)], wv)
                for cp_ in cps:
                    cp_.wait()

                def _mul(r, carry2):
                    wb = plsc.load_gather(wv, [jnp.full((16,), r, jnp.int32)])
                    for cc in range(cp // 16):
                        rows[r, pl.ds(cc * 16, 16)] = (
                            rows[r, pl.ds(cc * 16, 16)] * wb)
                    return carry2

                lax.fori_loop(0, CH, _mul, 0)
                for j in range(8):
                    pltpu.sync_copy(rows.at[pl.ds(j * 128, 128)],
                                    acc_sh.at[dstv.at[j]], add=True)
                return carry

            lax.fori_loop(0, nchunks, _chunk, 0)
            plsc.subcore_barrier()
            pltpu.sync_copy(
                acc_sh.at[pl.ds(sid * ROWS_SUB, ROWS_SUB)],
                out_hbm.at[h, cid, pl.ds(sid * ROWS_SUB, ROWS_SUB)])

    return sc_m


@functools.lru_cache(maxsize=None)
def _make_sc_g():
    ew = EPAD // NW
    nchunks = ew // CHG

    @functools.partial(
        pl.kernel,
        mesh=_MESH,
        out_type=jax.ShapeDtypeStruct((EPAD, 64), jnp.float32),
        scratch_types=[
            pltpu.VMEM((4, 128), jnp.int32),
            pltpu.VMEM((4, 128), jnp.int32),
            pltpu.VMEM((CHG, 64), jnp.float32),
            pltpu.VMEM((CHG, 64), jnp.float32),
            pltpu.SemaphoreType.DMA,
        ],
    )
    def sc_g(p_hbm, q_hbm, src_hbm, dst_hbm, s_hbm,
             srcv, dstv, prow, qrow, sem):
        cid = lax.axis_index("c")
        sid = lax.axis_index("s")
        base0 = (cid * 16 + sid) * ew

        def _chunk(ci, carry):
            ebase = base0 + ci * CHG
            eb128 = ebase // 128
            pltpu.sync_copy(src_hbm.at[pl.ds(eb128, 4)], srcv)
            pltpu.sync_copy(dst_hbm.at[pl.ds(eb128, 4)], dstv)
            cps = [pltpu.async_copy(p_hbm.at[srcv.at[j]],
                                    prow.at[pl.ds(j * 128, 128)], sem)
                   for j in range(4)]
            cps += [pltpu.async_copy(q_hbm.at[dstv.at[j]],
                                     qrow.at[pl.ds(j * 128, 128)], sem)
                    for j in range(4)]
            for cp in cps:
                cp.wait()

            def _add(r, carry2):
                for cc in range(4):
                    prow[r, pl.ds(cc * 16, 16)] = (
                        prow[r, pl.ds(cc * 16, 16)]
                        + qrow[r, pl.ds(cc * 16, 16)])
                return carry2

            lax.fori_loop(0, CHG, _add, 0)
            pltpu.sync_copy(prow, s_hbm.at[pl.ds(ebase, CHG)])
            return carry

        lax.fori_loop(0, nchunks, _chunk, 0)

    return sc_g


def _att_mat(a):
    eye = jnp.eye(H, dtype=jnp.float32)
    return (a[:, :, None] * eye[:, None, :]).reshape(-1, H)


def _gat_layer(h, src2d, dst2d, zeros16, zeros32, W, a_s, a_d, b, C,
               split_channels):
    xw, als, ald, maxs = _mmatt(h, W, _att_mat(a_s), _att_mat(a_d), C)
    at, slw = _assemble(als, ald, maxs)
    wt, asum2 = _make_sc_w()(at, src2d, dst2d, zeros16)
    if split_channels:
        xwt = xw.reshape(N * H * 2, 32)
        kmul = 2 * H
        zc = zeros32
    else:
        xwt = xw.reshape(N * H, C)
        kmul = H
        zc = zeros16 if C == 16 else zeros32
    acc = _make_sc_m(min(C, 32), split_channels, kmul)(
        xwt, src2d, dst2d, wt, zc)
    acc0 = acc[:, 0, :N, :]
    acc1 = acc[:, 1, :N, :]
    asum0 = asum2[0, :N, :H]
    asum1 = asum2[1, :N, :H]
    return _combine(acc0, acc1, asum0, asum1, slw, xw,
                    b.reshape(1, C), C, split_channels)


def kernel(x, edge_index, edge_attr, W1, a1s, a1d, b1, W2, a2s, a2d, b2,
           W3, a3s, a3d, b3, M1, mb1, M2, mb2, M3, mb3):
    src = edge_index[0]
    dst = edge_index[1]
    src2d = jnp.concatenate(
        [src, jnp.zeros((EPAD - E,), jnp.int32)]).reshape(EPAD // 128, 128)
    dst2d = jnp.concatenate(
        [dst, jnp.full((EPAD - E,), N, jnp.int32)]).reshape(EPAD // 128, 128)
    zeros16 = jnp.zeros((NPAD, 16), jnp.float32)
    zeros32 = jnp.zeros((NPAD, 32), jnp.float32)

    h = _gat_layer(x, src2d, dst2d, zeros16, zeros32,
                   W1, a1s, a1d, b1, 16, False)
    h = _gat_layer(h, src2d, dst2d, zeros16, zeros32,
                   W2, a2s, a2d, b2, 32, False)
    h = _gat_layer(h, src2d, dst2d, zeros16, zeros32,
                   W3, a3s, a3d, b3, 64, True)

    m1pq = jnp.concatenate([M1[:64], M1[64:128]], axis=1)
    p, q = _pq(h, m1pq)
    s = _make_sc_g()(p, q, src2d, dst2d)[:E]
    return _mlp(s, edge_attr, M1[128:], mb1.reshape(1, 64),
                M2, mb2.reshape(1, 16), M3, mb3.reshape(1, 1))


# speedup vs baseline: 8.1045x; 8.1045x over previous
import functools

import jax
import jax.numpy as jnp
from jax import lax
from jax.experimental import pallas as pl
from jax.experimental.pallas import tpu as pltpu
from jax.experimental.pallas import tpu_sc as plsc

N = 50000
E = 800000
H = 4
EPAD = 819200
NW = 32
CHD = 128
BN = 2000
BE = 2000


def _mmatt_body(h_ref, w_ref, as_ref, ad_ref, xw_ref, als_ref, ald_ref,
                maxs_ref):
    i = pl.program_id(0)
    xw = jnp.dot(h_ref[...], w_ref[...], preferred_element_type=jnp.float32)
    xw_ref[...] = xw
    als = jnp.dot(xw, as_ref[...], preferred_element_type=jnp.float32)
    ald = jnp.dot(xw, ad_ref[...], preferred_element_type=jnp.float32)
    als_ref[...] = als
    ald_ref[...] = ald

    @pl.when(i == 0)
    def _():
        maxs_ref[...] = jnp.full_like(maxs_ref[...], -jnp.inf)

    maxs_ref[...] = jnp.maximum(maxs_ref[...],
                                jnp.max(als, axis=0, keepdims=True))


def _mmatt(h, W, as_mat, ad_mat, C):
    n_in = h.shape[1]
    return pl.pallas_call(
        _mmatt_body,
        grid=(N // BN,),
        in_specs=[
            pl.BlockSpec((BN, n_in), lambda i: (i, 0)),
            pl.BlockSpec((n_in, H * C), lambda i: (0, 0)),
            pl.BlockSpec((H * C, H), lambda i: (0, 0)),
            pl.BlockSpec((H * C, H), lambda i: (0, 0)),
        ],
        out_specs=[
            pl.BlockSpec((BN, H * C), lambda i: (i, 0)),
            pl.BlockSpec((BN, H), lambda i: (i, 0)),
            pl.BlockSpec((BN, H), lambda i: (i, 0)),
            pl.BlockSpec((1, H), lambda i: (0, 0)),
        ],
        out_shape=[
            jax.ShapeDtypeStruct((N, H * C), jnp.float32),
            jax.ShapeDtypeStruct((N, H), jnp.float32),
            jax.ShapeDtypeStruct((N, H), jnp.float32),
            jax.ShapeDtypeStruct((1, H), jnp.float32),
        ],
    )(h, W, as_mat, ad_mat)


def _assemble_body(als_ref, ald_ref, maxs_ref, ats_ref, atd_ref, slw_ref):
    als = als_ref[...]
    ald = ald_ref[...]
    ms = maxs_ref[...]
    t = ms + ald
    bnd = jnp.maximum(t, 0.2 * t)
    a = als + ald
    a = jnp.maximum(a, 0.2 * a)
    slw_ref[...] = jnp.exp(a - bnd)
    z12 = jnp.zeros((BN, 12), jnp.float32)
    z108 = jnp.zeros((BN, 108), jnp.float32)
    z124 = jnp.zeros((BN, 124), jnp.float32)
    ats_ref[...] = jnp.concatenate([als, z124], axis=1)
    atd_ref[...] = jnp.concatenate([ald, z12, bnd, z108], axis=1)


def _assemble(als, ald, maxs):
    return pl.pallas_call(
        _assemble_body,
        grid=(N // BN,),
        in_specs=[
            pl.BlockSpec((BN, H), lambda i: (i, 0)),
            pl.BlockSpec((BN, H), lambda i: (i, 0)),
            pl.BlockSpec((1, H), lambda i: (0, 0)),
        ],
        out_specs=[
            pl.BlockSpec((BN, 128), lambda i: (i, 0)),
            pl.BlockSpec((BN, 128), lambda i: (i, 0)),
            pl.BlockSpec((BN, H), lambda i: (i, 0)),
        ],
        out_shape=[
            jax.ShapeDtypeStruct((N, 128), jnp.float32),
            jax.ShapeDtypeStruct((N, 128), jnp.float32),
            jax.ShapeDtypeStruct((N, H), jnp.float32),
        ],
    )(als, ald, maxs)


def _acc_body(msg_ref, we_ref, dst_ref, acc_ref, cat_ref, *, has_w):
    i = pl.program_id(0)

    @pl.when(i == 0)
    def _():
        acc_ref[...] = jnp.zeros(acc_ref.shape, jnp.float32)

    if has_w:
        cat_ref[...] = jnp.concatenate([msg_ref[...], we_ref[...]], axis=1)
    else:
        cat_ref[...] = msg_ref[...]

    def _edge(e, carry):
        d = dst_ref[0, 0, e]
        acc_ref[pl.ds(d, 1), :] = (acc_ref[pl.ds(d, 1), :]
                                   + cat_ref[pl.ds(e, 1), :])
        return carry

    lax.fori_loop(0, BE, _edge, 0)


def _acc(msg, we, dst2d, has_w):
    mw = msg.shape[1]
    width = mw + (16 if has_w else 0)
    body = functools.partial(_acc_body, has_w=has_w)
    in_specs = [
        pl.BlockSpec((BE, mw), lambda i: (i, 0)),
        pl.BlockSpec((BE, 16), lambda i: (i, 0)),
        pl.BlockSpec((1, 1, BE), lambda i: (i, 0, 0), memory_space=pltpu.SMEM),
    ]
    return pl.pallas_call(
        body,
        grid=(E // BE,),
        in_specs=in_specs,
        out_specs=pl.BlockSpec((N, width), lambda i: (0, 0)),
        out_shape=jax.ShapeDtypeStruct((N, width), jnp.float32),
        scratch_shapes=[pltpu.VMEM((BE, width), jnp.float32)],
    )(msg, we, dst2d)


def _combine_body(acc0_ref, acc1_ref, slw_ref, xw_ref, b_ref, out_ref, *, C):
    slw = slw_ref[...]
    out = jnp.zeros(out_ref.shape, jnp.float32)
    for h in range(H):
        if C == 64:
            if h < 2:
                num = acc0_ref[:, h * 64:(h + 1) * 64]
            else:
                num = acc1_ref[:, (h - 2) * 64:(h - 1) * 64]
        else:
            num = acc0_ref[:, h * C:(h + 1) * C]
        wcol = acc0_ref[:, 128 + h:129 + h]
        num = num + slw[:, h:h + 1] * xw_ref[:, h * C:(h + 1) * C]
        den = wcol + slw[:, h:h + 1] + 1e-16
        out = out + num / den
    out_ref[...] = out * (1.0 / H) + b_ref[...]


def _combine(acc0, acc1, slw, xw, bias, C):
    body = functools.partial(_combine_body, C=C)
    w1 = acc1.shape[1]
    return pl.pallas_call(
        body,
        grid=(N // BN,),
        in_specs=[
            pl.BlockSpec((BN, 144), lambda i: (i, 0)),
            pl.BlockSpec((BN, w1), lambda i: (i, 0)),
            pl.BlockSpec((BN, H), lambda i: (i, 0)),
            pl.BlockSpec((BN, H * C), lambda i: (i, 0)),
            pl.BlockSpec((1, C), lambda i: (0, 0)),
        ],
        out_specs=pl.BlockSpec((BN, C), lambda i: (i, 0)),
        out_shape=jax.ShapeDtypeStruct((N, C), jnp.float32),
    )(acc0, acc1, slw, xw, bias)


def _pq_body(h_ref, m_ref, pq_ref):
    pq_ref[...] = jnp.dot(h_ref[...], m_ref[...],
                          preferred_element_type=jnp.float32)


def _pq(h3, m1pq):
    return pl.pallas_call(
        _pq_body,
        grid=(N // BN,),
        in_specs=[
            pl.BlockSpec((BN, 64), lambda i: (i, 0)),
            pl.BlockSpec((64, 128), lambda i: (0, 0)),
        ],
        out_specs=pl.BlockSpec((BN, 128), lambda i: (i, 0)),
        out_shape=jax.ShapeDtypeStruct((N, 128), jnp.float32),
    )(h3, m1pq)


def _mlp_body(s_ref, ea_ref, m1c_ref, mb1_ref, m2_ref, mb2_ref, m3_ref,
              mb3_ref, out_ref):
    z = (s_ref[...]
         + jnp.dot(ea_ref[...], m1c_ref[...],
                   preferred_element_type=jnp.float32)
         + mb1_ref[...])
    z = jnp.maximum(z, 0.12 * z)
    z = jnp.dot(z, m2_ref[...], preferred_element_type=jnp.float32) + mb2_ref[...]
    z = jnp.maximum(z, 0.12 * z)
    t = jnp.dot(z, m3_ref[...], preferred_element_type=jnp.float32) + mb3_ref[...]
    out_ref[...] = jax.nn.sigmoid(t)


def _mlp(s, ea, m1c, mb1, m2, mb2, m3, mb3):
    return pl.pallas_call(
        _mlp_body,
        grid=(E // BE,),
        in_specs=[
            pl.BlockSpec((BE, 64), lambda i: (i, 0)),
            pl.BlockSpec((BE, 10), lambda i: (i, 0)),
            pl.BlockSpec((10, 64), lambda i: (0, 0)),
            pl.BlockSpec((1, 64), lambda i: (0, 0)),
            pl.BlockSpec((64, 16), lambda i: (0, 0)),
            pl.BlockSpec((1, 16), lambda i: (0, 0)),
            pl.BlockSpec((16, 1), lambda i: (0, 0)),
            pl.BlockSpec((1, 1), lambda i: (0, 0)),
        ],
        out_specs=pl.BlockSpec((BE, 1), lambda i: (i, 0)),
        out_shape=jax.ShapeDtypeStruct((E, 1), jnp.float32),
    )(s, ea, m1c, mb1, m2, mb2, m3, mb3)


_MESH = plsc.VectorSubcoreMesh(core_axis_name="c", subcore_axis_name="s")
_EW = EPAD // NW
_NCHUNK = _EW // CHD


@functools.lru_cache(maxsize=None)
def _make_sc_edge(parts, C):
    @functools.partial(
        pl.kernel,
        mesh=_MESH,
        out_type=[
            jax.ShapeDtypeStruct((EPAD, 16), jnp.float32),
            jax.ShapeDtypeStruct((parts, EPAD, 128), jnp.float32),
        ],
        scratch_types=[
            pltpu.VMEM((CHD,), jnp.int32),
            pltpu.VMEM((CHD,), jnp.int32),
            pltpu.VMEM((2, CHD), jnp.int32),
            pltpu.VMEM((CHD, 128), jnp.float32),
            pltpu.VMEM((CHD, 128), jnp.float32),
            pltpu.VMEM((parts, CHD, 128), jnp.float32),
            pltpu.VMEM((CHD, 16), jnp.float32),
            pltpu.SemaphoreType.DMA,
        ],
    )
    def sc_edge(ats_hbm, atd_hbm, xwp_hbm, src_hbm, dst_hbm, we_hbm, msg_hbm,
                sidx, didx, gidx, srow, drow, mrow, wbuf, sem):
        cid = lax.axis_index("c")
        sid = lax.axis_index("s")
        base0 = (cid * 16 + sid) * _EW
        lane = lax.iota(jnp.int32, 16)

        def _chunk(ci, carry):
            ebase = pl.multiple_of(base0 + ci * CHD, 8)
            pltpu.sync_copy(src_hbm.at[pl.ds(ebase, CHD)], sidx)
            pltpu.sync_copy(dst_hbm.at[pl.ds(ebase, CHD)], didx)
            cps = [pltpu.async_copy(ats_hbm.at[sidx], srow, sem),
                   pltpu.async_copy(atd_hbm.at[didx], drow, sem)]
            for p in range(parts):
                if parts > 1:
                    def _gi(t, carry2, p=p):
                        v = sidx[pl.ds(t * 16, 16)]
                        gidx[p, pl.ds(t * 16, 16)] = v * parts + p
                        return carry2

                    lax.fori_loop(0, CHD // 16, _gi, 0)
                    cps.append(pltpu.async_copy(xwp_hbm.at[gidx.at[p]],
                                                mrow.at[p], sem))
                else:
                    cps.append(pltpu.async_copy(xwp_hbm.at[sidx],
                                                mrow.at[p], sem))
            for cp in cps:
                cp.wait()

            def _row(r, carry2):
                vs = srow[r, pl.ds(0, 16)]
                vd = drow[r, pl.ds(0, 16)]
                vb = drow[r, pl.ds(16, 16)]
                a = vs + vd
                a = jnp.maximum(a, 0.2 * a)
                w = jnp.exp(a - vb)
                w = jnp.where(lane < H, w, 0.0)
                wbuf[r, :] = w
                for p in range(parts):
                    for hl in range(H // parts):
                        h = p * (H // parts) + hl
                        wb = jnp.zeros((16,), jnp.float32) + w[h]
                        for k in range(C // 16):
                            sl = pl.ds(hl * C + k * 16, 16)
                            mrow[p, r, sl] = mrow[p, r, sl] * wb
                return carry2

            lax.fori_loop(0, CHD, _row, 0)
            pltpu.sync_copy(wbuf, we_hbm.at[pl.ds(ebase, CHD)])
            for p in range(parts):
                pltpu.sync_copy(mrow.at[p],
                                msg_hbm.at[p, pl.ds(ebase, CHD)])
            return carry

        lax.fori_loop(0, _NCHUNK, _chunk, 0)

    return sc_edge


@functools.lru_cache(maxsize=None)
def _make_sc_pq():
    @functools.partial(
        pl.kernel,
        mesh=_MESH,
        out_type=jax.ShapeDtypeStruct((EPAD, 64), jnp.float32),
        scratch_types=[
            pltpu.VMEM((CHD,), jnp.int32),
            pltpu.VMEM((CHD,), jnp.int32),
            pltpu.VMEM((CHD, 128), jnp.float32),
            pltpu.VMEM((CHD, 128), jnp.float32),
            pltpu.VMEM((CHD, 64), jnp.float32),
            pltpu.SemaphoreType.DMA,
        ],
    )
    def sc_pq(pq_hbm, src_hbm, dst_hbm, s_hbm, sidx, didx, srow, drow, sbuf,
              sem):
        cid = lax.axis_index("c")
        sid = lax.axis_index("s")
        base0 = (cid * 16 + sid) * _EW

        def _chunk(ci, carry):
            ebase = pl.multiple_of(base0 + ci * CHD, 8)
            pltpu.sync_copy(src_hbm.at[pl.ds(ebase, CHD)], sidx)
            pltpu.sync_copy(dst_hbm.at[pl.ds(ebase, CHD)], didx)
            cps = [pltpu.async_copy(pq_hbm.at[sidx], srow, sem),
                   pltpu.async_copy(pq_hbm.at[didx], drow, sem)]
            for cp in cps:
                cp.wait()

            def _row(r, carry2):
                for k in range(4):
                    sbuf[r, pl.ds(k * 16, 16)] = (
                        srow[r, pl.ds(k * 16, 16)]
                        + drow[r, pl.ds(64 + k * 16, 16)])
                return carry2

            lax.fori_loop(0, CHD, _row, 0)
            pltpu.sync_copy(sbuf, s_hbm.at[pl.ds(ebase, CHD)])
            return carry

        lax.fori_loop(0, _NCHUNK, _chunk, 0)

    return sc_pq


def _att_mat(a):
    eye = jnp.eye(H, dtype=jnp.float32)
    return (a[:, :, None] * eye[:, None, :]).reshape(-1, H)


def _gat_layer(h, src1, dst1, dst2d, W, a_s, a_d, b, C):
    xw, als, ald, maxs = _mmatt(h, W, _att_mat(a_s), _att_mat(a_d), C)
    ats, atd, slw = _assemble(als, ald, maxs)
    if C == 16:
        xwp = jnp.concatenate([xw, jnp.zeros((N, 64), jnp.float32)], axis=1)
        parts = 1
    elif C == 32:
        xwp = xw
        parts = 1
    else:
        xwp = xw.reshape(2 * N, 128)
        parts = 2
    we, msg = _make_sc_edge(parts, C)(ats, atd, xwp, src1, dst1)
    acc0 = _acc(msg[0, :E], we[:E], dst2d, True)
    if parts == 2:
        acc1 = _acc(msg[1, :E], we[:E], dst2d, False)
    else:
        acc1 = acc0
    return _combine(acc0, acc1, slw, xw, b.reshape(1, C), C)


def kernel(x, edge_index, edge_attr, W1, a1s, a1d, b1, W2, a2s, a2d, b2,
           W3, a3s, a3d, b3, M1, mb1, M2, mb2, M3, mb3):
    src = edge_index[0]
    dst = edge_index[1]
    src1 = jnp.concatenate([src, jnp.zeros((EPAD - E,), jnp.int32)])
    dst1 = jnp.concatenate([dst, jnp.zeros((EPAD - E,), jnp.int32)])
    dst2d = dst.reshape(E // BE, 1, BE)

    h = _gat_layer(x, src1, dst1, dst2d, W1, a1s, a1d, b1, 16)
    h = _gat_layer(h, src1, dst1, dst2d, W2, a2s, a2d, b2, 32)
    h = _gat_layer(h, src1, dst1, dst2d, W3, a3s, a3d, b3, 64)

    m1pq = jnp.concatenate([M1[:64], M1[64:128]], axis=1)
    pq = _pq(h, m1pq)
    s = _make_sc_pq()(pq, src1, dst1)[:E]
    return _mlp(s, edge_attr, M1[128:], mb1.reshape(1, 64),
                M2, mb2.reshape(1, 16), M3, mb3.reshape(1, 1))


# speedup vs baseline: 10.3096x; 1.2721x over previous
import functools

import jax
import jax.numpy as jnp
from jax import lax
from jax.experimental import pallas as pl
from jax.experimental.pallas import tpu as pltpu
from jax.experimental.pallas import tpu_sc as plsc

N = 50000
E = 800000
H = 4
EPAD = 819200
NW = 32
CHD = 128
BN = 2000
BE = 2000


def _mmatt_body(h_ref, w_ref, as_ref, ad_ref, xw_ref, als_ref, ald_ref,
                maxs_ref):
    i = pl.program_id(0)
    xw = jnp.dot(h_ref[...], w_ref[...], preferred_element_type=jnp.float32)
    xw_ref[...] = xw
    als = jnp.dot(xw, as_ref[...], preferred_element_type=jnp.float32)
    ald = jnp.dot(xw, ad_ref[...], preferred_element_type=jnp.float32)
    als_ref[...] = als
    ald_ref[...] = ald

    @pl.when(i == 0)
    def _():
        maxs_ref[...] = jnp.full_like(maxs_ref[...], -jnp.inf)

    maxs_ref[...] = jnp.maximum(maxs_ref[...],
                                jnp.max(als, axis=0, keepdims=True))


def _mmatt(h, W, as_mat, ad_mat, C):
    n_in = h.shape[1]
    return pl.pallas_call(
        _mmatt_body,
        grid=(N // BN,),
        in_specs=[
            pl.BlockSpec((BN, n_in), lambda i: (i, 0)),
            pl.BlockSpec((n_in, H * C), lambda i: (0, 0)),
            pl.BlockSpec((H * C, H), lambda i: (0, 0)),
            pl.BlockSpec((H * C, H), lambda i: (0, 0)),
        ],
        out_specs=[
            pl.BlockSpec((BN, H * C), lambda i: (i, 0)),
            pl.BlockSpec((BN, H), lambda i: (i, 0)),
            pl.BlockSpec((BN, H), lambda i: (i, 0)),
            pl.BlockSpec((1, H), lambda i: (0, 0)),
        ],
        out_shape=[
            jax.ShapeDtypeStruct((N, H * C), jnp.float32),
            jax.ShapeDtypeStruct((N, H), jnp.float32),
            jax.ShapeDtypeStruct((N, H), jnp.float32),
            jax.ShapeDtypeStruct((1, H), jnp.float32),
        ],
    )(h, W, as_mat, ad_mat)


def _assemble_body(als_ref, ald_ref, maxs_ref, ats_ref, atd_ref, slw_ref):
    als = als_ref[...]
    ald = ald_ref[...]
    ms = maxs_ref[...]
    t = ms + ald
    bnd = jnp.maximum(t, 0.2 * t)
    a = als + ald
    a = jnp.maximum(a, 0.2 * a)
    slw_ref[...] = jnp.exp(a - bnd)
    z12 = jnp.zeros((BN, 12), jnp.float32)
    z108 = jnp.zeros((BN, 108), jnp.float32)
    z124 = jnp.zeros((BN, 124), jnp.float32)
    ats_ref[...] = jnp.concatenate([als, z124], axis=1)
    atd_ref[...] = jnp.concatenate([ald, z12, bnd, z108], axis=1)


def _assemble(als, ald, maxs):
    return pl.pallas_call(
        _assemble_body,
        grid=(N // BN,),
        in_specs=[
            pl.BlockSpec((BN, H), lambda i: (i, 0)),
            pl.BlockSpec((BN, H), lambda i: (i, 0)),
            pl.BlockSpec((1, H), lambda i: (0, 0)),
        ],
        out_specs=[
            pl.BlockSpec((BN, 128), lambda i: (i, 0)),
            pl.BlockSpec((BN, 128), lambda i: (i, 0)),
            pl.BlockSpec((BN, H), lambda i: (i, 0)),
        ],
        out_shape=[
            jax.ShapeDtypeStruct((N, 128), jnp.float32),
            jax.ShapeDtypeStruct((N, 128), jnp.float32),
            jax.ShapeDtypeStruct((N, H), jnp.float32),
        ],
    )(als, ald, maxs)


def _acc_body(msg_ref, we_ref, dst_ref, acc_ref, cat_ref, *, has_w):
    i = pl.program_id(0)

    @pl.when(i == 0)
    def _():
        acc_ref[...] = jnp.zeros(acc_ref.shape, jnp.float32)

    if has_w:
        cat_ref[...] = jnp.concatenate([msg_ref[...], we_ref[...]], axis=1)
    else:
        cat_ref[...] = msg_ref[...]

    def _edge(e4, carry):
        for u in range(4):
            e = e4 * 4 + u
            d = dst_ref[0, 0, e]
            acc_ref[pl.ds(d, 1), :] = (acc_ref[pl.ds(d, 1), :]
                                       + cat_ref[pl.ds(e, 1), :])
        return carry

    lax.fori_loop(0, BE // 4, _edge, 0)


def _acc(msg, we, dst2d, has_w):
    mw = msg.shape[1]
    width = mw + (16 if has_w else 0)
    body = functools.partial(_acc_body, has_w=has_w)
    in_specs = [
        pl.BlockSpec((BE, mw), lambda i: (i, 0)),
        pl.BlockSpec((BE, 16), lambda i: (i, 0)),
        pl.BlockSpec((1, 1, BE), lambda i: (i, 0, 0), memory_space=pltpu.SMEM),
    ]
    return pl.pallas_call(
        body,
        grid=(E // BE,),
        in_specs=in_specs,
        out_specs=pl.BlockSpec((N, width), lambda i: (0, 0)),
        out_shape=jax.ShapeDtypeStruct((N, width), jnp.float32),
        scratch_shapes=[pltpu.VMEM((BE, width), jnp.float32)],
    )(msg, we, dst2d)


def _combine_body(acc0_ref, acc1_ref, slw_ref, xw_ref, b_ref, out_ref, *, C):
    slw = slw_ref[...]
    out = jnp.zeros(out_ref.shape, jnp.float32)
    for h in range(H):
        if C == 64:
            if h < 2:
                num = acc0_ref[:, h * 64:(h + 1) * 64]
            else:
                num = acc1_ref[:, (h - 2) * 64:(h - 1) * 64]
        else:
            num = acc0_ref[:, h * C:(h + 1) * C]
        wcol = acc0_ref[:, 128 + h:129 + h]
        num = num + slw[:, h:h + 1] * xw_ref[:, h * C:(h + 1) * C]
        den = wcol + slw[:, h:h + 1] + 1e-16
        out = out + num / den
    out_ref[...] = out * (1.0 / H) + b_ref[...]


def _combine(acc0, acc1, slw, xw, bias, C):
    body = functools.partial(_combine_body, C=C)
    w1 = acc1.shape[1]
    return pl.pallas_call(
        body,
        grid=(N // BN,),
        in_specs=[
            pl.BlockSpec((BN, 144), lambda i: (i, 0)),
            pl.BlockSpec((BN, w1), lambda i: (i, 0)),
            pl.BlockSpec((BN, H), lambda i: (i, 0)),
            pl.BlockSpec((BN, H * C), lambda i: (i, 0)),
            pl.BlockSpec((1, C), lambda i: (0, 0)),
        ],
        out_specs=pl.BlockSpec((BN, C), lambda i: (i, 0)),
        out_shape=jax.ShapeDtypeStruct((N, C), jnp.float32),
    )(acc0, acc1, slw, xw, bias)


def _pq_body(h_ref, m_ref, pq_ref):
    pq_ref[...] = jnp.dot(h_ref[...], m_ref[...],
                          preferred_element_type=jnp.float32)


def _pq(h3, m1pq):
    return pl.pallas_call(
        _pq_body,
        grid=(N // BN,),
        in_specs=[
            pl.BlockSpec((BN, 64), lambda i: (i, 0)),
            pl.BlockSpec((64, 128), lambda i: (0, 0)),
        ],
        out_specs=pl.BlockSpec((BN, 128), lambda i: (i, 0)),
        out_shape=jax.ShapeDtypeStruct((N, 128), jnp.float32),
    )(h3, m1pq)


def _mlp_body(s_ref, ea_ref, m1c_ref, mb1_ref, m2_ref, mb2_ref, m3_ref,
              mb3_ref, out_ref):
    z = (s_ref[...]
         + jnp.dot(ea_ref[...], m1c_ref[...],
                   preferred_element_type=jnp.float32)
         + mb1_ref[...])
    z = jnp.maximum(z, 0.12 * z)
    z = jnp.dot(z, m2_ref[...], preferred_element_type=jnp.float32) + mb2_ref[...]
    z = jnp.maximum(z, 0.12 * z)
    t = jnp.dot(z, m3_ref[...], preferred_element_type=jnp.float32) + mb3_ref[...]
    out_ref[...] = jax.nn.sigmoid(t)


def _mlp(s, ea, m1c, mb1, m2, mb2, m3, mb3):
    return pl.pallas_call(
        _mlp_body,
        grid=(E // BE,),
        in_specs=[
            pl.BlockSpec((BE, 64), lambda i: (i, 0)),
            pl.BlockSpec((BE, 10), lambda i: (i, 0)),
            pl.BlockSpec((10, 64), lambda i: (0, 0)),
            pl.BlockSpec((1, 64), lambda i: (0, 0)),
            pl.BlockSpec((64, 16), lambda i: (0, 0)),
            pl.BlockSpec((1, 16), lambda i: (0, 0)),
            pl.BlockSpec((16, 1), lambda i: (0, 0)),
            pl.BlockSpec((1, 1), lambda i: (0, 0)),
        ],
        out_specs=pl.BlockSpec((BE, 1), lambda i: (i, 0)),
        out_shape=jax.ShapeDtypeStruct((E, 1), jnp.float32),
    )(s, ea, m1c, mb1, m2, mb2, m3, mb3)


_MESH = plsc.VectorSubcoreMesh(core_axis_name="c", subcore_axis_name="s")
_EW = EPAD // NW
_NCHUNK = _EW // CHD


@functools.lru_cache(maxsize=None)
def _make_sc_edge(parts, C):
    @functools.partial(
        pl.kernel,
        mesh=_MESH,
        out_type=[
            jax.ShapeDtypeStruct((EPAD, 16), jnp.float32),
            jax.ShapeDtypeStruct((parts, EPAD, 128), jnp.float32),
        ],
        scratch_types=[
            pltpu.VMEM((CHD,), jnp.int32),
            pltpu.VMEM((CHD,), jnp.int32),
            pltpu.VMEM((2, CHD), jnp.int32),
            pltpu.VMEM((CHD, 128), jnp.float32),
            pltpu.VMEM((CHD, 128), jnp.float32),
            pltpu.VMEM((parts, CHD, 128), jnp.float32),
            pltpu.VMEM((CHD, 16), jnp.float32),
            pltpu.SemaphoreType.DMA,
        ],
    )
    def sc_edge(ats_hbm, atd_hbm, xwp_hbm, src_hbm, dst_hbm, we_hbm, msg_hbm,
                sidx, didx, gidx, srow, drow, mrow, wbuf, sem):
        cid = lax.axis_index("c")
        sid = lax.axis_index("s")
        base0 = (cid * 16 + sid) * _EW
        lane = lax.iota(jnp.int32, 16)

        def _chunk(ci, carry):
            ebase = pl.multiple_of(base0 + ci * CHD, 8)
            pltpu.sync_copy(src_hbm.at[pl.ds(ebase, CHD)], sidx)
            pltpu.sync_copy(dst_hbm.at[pl.ds(ebase, CHD)], didx)
            cps = [pltpu.async_copy(ats_hbm.at[sidx], srow, sem),
                   pltpu.async_copy(atd_hbm.at[didx], drow, sem)]
            for p in range(parts):
                if parts > 1:
                    def _gi(t, carry2, p=p):
                        v = sidx[pl.ds(t * 16, 16)]
                        gidx[p, pl.ds(t * 16, 16)] = v * parts + p
                        return carry2

                    lax.fori_loop(0, CHD // 16, _gi, 0)
                    cps.append(pltpu.async_copy(xwp_hbm.at[gidx.at[p]],
                                                mrow.at[p], sem))
                else:
                    cps.append(pltpu.async_copy(xwp_hbm.at[sidx],
                                                mrow.at[p], sem))
            for cp in cps:
                cp.wait()

            def _row(r, carry2):
                vs = srow[r, pl.ds(0, 16)]
                vd = drow[r, pl.ds(0, 16)]
                vb = drow[r, pl.ds(16, 16)]
                a = vs + vd
                a = jnp.maximum(a, 0.2 * a)
                w = jnp.exp(a - vb)
                w = jnp.where(lane < H, w, 0.0)
                wbuf[r, :] = w
                for p in range(parts):
                    for hl in range(H // parts):
                        h = p * (H // parts) + hl
                        wb = jnp.zeros((16,), jnp.float32) + w[h]
                        for k in range(C // 16):
                            sl = pl.ds(hl * C + k * 16, 16)
                            mrow[p, r, sl] = mrow[p, r, sl] * wb
                return carry2

            lax.fori_loop(0, CHD, _row, 0)
            pltpu.sync_copy(wbuf, we_hbm.at[pl.ds(ebase, CHD)])
            for p in range(parts):
                pltpu.sync_copy(mrow.at[p],
                                msg_hbm.at[p, pl.ds(ebase, CHD)])
            return carry

        lax.fori_loop(0, _NCHUNK, _chunk, 0)

    return sc_edge


@functools.lru_cache(maxsize=None)
def _make_sc_pq():
    @functools.partial(
        pl.kernel,
        mesh=_MESH,
        out_type=jax.ShapeDtypeStruct((EPAD, 64), jnp.float32),
        scratch_types=[
            pltpu.VMEM((CHD,), jnp.int32),
            pltpu.VMEM((CHD,), jnp.int32),
            pltpu.VMEM((CHD, 128), jnp.float32),
            pltpu.VMEM((CHD, 128), jnp.float32),
            pltpu.VMEM((CHD, 64), jnp.float32),
            pltpu.SemaphoreType.DMA,
        ],
    )
    def sc_pq(pq_hbm, src_hbm, dst_hbm, s_hbm, sidx, didx, srow, drow, sbuf,
              sem):
        cid = lax.axis_index("c")
        sid = lax.axis_index("s")
        base0 = (cid * 16 + sid) * _EW

        def _chunk(ci, carry):
            ebase = pl.multiple_of(base0 + ci * CHD, 8)
            pltpu.sync_copy(src_hbm.at[pl.ds(ebase, CHD)], sidx)
            pltpu.sync_copy(dst_hbm.at[pl.ds(ebase, CHD)], didx)
            cps = [pltpu.async_copy(pq_hbm.at[sidx], srow, sem),
                   pltpu.async_copy(pq_hbm.at[didx], drow, sem)]
            for cp in cps:
                cp.wait()

            def _row(r, carry2):
                for k in range(4):
                    sbuf[r, pl.ds(k * 16, 16)] = (
                        srow[r, pl.ds(k * 16, 16)]
                        + drow[r, pl.ds(64 + k * 16, 16)])
                return carry2

            lax.fori_loop(0, CHD, _row, 0)
            pltpu.sync_copy(sbuf, s_hbm.at[pl.ds(ebase, CHD)])
            return carry

        lax.fori_loop(0, _NCHUNK, _chunk, 0)

    return sc_pq


def _att_mat(a):
    eye = jnp.eye(H, dtype=jnp.float32)
    return (a[:, :, None] * eye[:, None, :]).reshape(-1, H)


def _gat_layer(h, src1, dst1, dst2d, W, a_s, a_d, b, C):
    xw, als, ald, maxs = _mmatt(h, W, _att_mat(a_s), _att_mat(a_d), C)
    ats, atd, slw = _assemble(als, ald, maxs)
    if C == 16:
        xwp = jnp.concatenate([xw, jnp.zeros((N, 64), jnp.float32)], axis=1)
        parts = 1
    elif C == 32:
        xwp = xw
        parts = 1
    else:
        xwp = xw.reshape(2 * N, 128)
        parts = 2
    we, msg = _make_sc_edge(parts, C)(ats, atd, xwp, src1, dst1)
    acc0 = _acc(msg[0, :E], we[:E], dst2d, True)
    if parts == 2:
        acc1 = _acc(msg[1, :E], we[:E], dst2d, False)
    else:
        acc1 = acc0
    return _combine(acc0, acc1, slw, xw, b.reshape(1, C), C)


def kernel(x, edge_index, edge_attr, W1, a1s, a1d, b1, W2, a2s, a2d, b2,
           W3, a3s, a3d, b3, M1, mb1, M2, mb2, M3, mb3):
    src = edge_index[0]
    dst = edge_index[1]
    src1 = jnp.concatenate([src, jnp.zeros((EPAD - E,), jnp.int32)])
    dst1 = jnp.concatenate([dst, jnp.zeros((EPAD - E,), jnp.int32)])
    dst2d = dst.reshape(E // BE, 1, BE)

    h = _gat_layer(x, src1, dst1, dst2d, W1, a1s, a1d, b1, 16)
    h = _gat_layer(h, src1, dst1, dst2d, W2, a2s, a2d, b2, 32)
    h = _gat_layer(h, src1, dst1, dst2d, W3, a3s, a3d, b3, 64)

    m1pq = jnp.concatenate([M1[:64], M1[64:128]], axis=1)
    pq = _pq(h, m1pq)
    s = _make_sc_pq()(pq, src1, dst1)[:E]
    return _mlp(s, edge_attr, M1[128:], mb1.reshape(1, 64),
                M2, mb2.reshape(1, 16), M3, mb3.reshape(1, 1))
